# baseline (speedup 1.0000x reference)
import jax
import jax.numpy as jnp
import numpy as np
from jax import lax
from jax.experimental import pallas as pl
from jax.experimental.pallas import tpu as pltpu
from jax.experimental.pallas import tpu_sc as plsc

_N_VOX = (200, 200, 12)
_VOXEL = np.array([0.5, 0.5, 8.0 / 12.0], dtype=np.float32)
_ORIGIN = np.array([0.0, 0.0, -1.0], dtype=np.float32)

_NCAM, _C, _H, _W = 6, 64, 64, 176
_NX, _NY, _NZ = _N_VOX
_P = _NX * _NY * _NZ
_NROWS = _NCAM * _H * _W
_ZROW = _NROWS
_TBL_ROWS = _NROWS + 8

_NC, _NS = 2, 16
_NW = _NC * _NS
_BPW = _P // _NW
_CHUNK = 60
_NCH = _BPW // _CHUNK
_GRP = 10
_NGRP = _NCH // _GRP

_QB = 19200
_NB = _P // _QB


def _points_zyx():
    zs = jnp.arange(_NZ, dtype=jnp.float32)
    ys = jnp.arange(_NY, dtype=jnp.float32)
    xs = jnp.arange(_NX, dtype=jnp.float32)
    zz, yy, xx = jnp.meshgrid(zs, ys, xs, indexing='ij')
    pts = jnp.stack([xx, yy, zz])
    vs = jnp.asarray(_VOXEL)
    new_origin = (jnp.asarray(_ORIGIN)
                  - jnp.asarray(_N_VOX, dtype=jnp.float32) / 2.0 * vs)
    return pts * vs.reshape(3, 1, 1, 1) + new_origin.reshape(3, 1, 1, 1)


def _sel_body(p_ref, idx_ref):
    s = jnp.full((1, _QB), _ZROW, dtype=jnp.int32)
    for i in range(_NCAM):
        px = p_ref[3 * i:3 * i + 1, :]
        py = p_ref[3 * i + 1:3 * i + 2, :]
        pz = p_ref[3 * i + 2:3 * i + 3, :]
        xi = jnp.round(px / pz).astype(jnp.int32)
        yi = jnp.round(py / pz).astype(jnp.int32)
        valid = ((xi >= 0) & (yi >= 0) & (xi < _W) & (yi < _H)
                 & (pz > 0.0))
        row = (yi * _W + xi) + i * (_H * _W)
        s = jnp.where(valid, row, s)
    idx_ref[...] = s[None]


def _select_rows(p2):
    return pl.pallas_call(
        _sel_body,
        grid=(_NB,),
        in_specs=[pl.BlockSpec((_NCAM * 3, _QB), lambda i: (0, i))],
        out_specs=pl.BlockSpec((1, 1, _QB), lambda i: (i, 0, 0)),
        out_shape=jax.ShapeDtypeStruct((_NB, 1, _QB), jnp.int32),
    )(p2)


def _sc_gather_body(table_hbm, idx_hbm, out_hbm, idx_v, *rest):
    bufs = rest[:_GRP]
    gsems = rest[_GRP:2 * _GRP]
    wsems = rest[2 * _GRP:]
    wid = lax.axis_index("s") * _NC + lax.axis_index("c")
    base = wid * _BPW

    pltpu.sync_copy(idx_hbm.at[wid], idx_v)

    for b in range(_GRP):
        pltpu.async_copy(table_hbm.at[idx_v.at[b]], bufs[b], gsems[b])

    def body(g, carry):
        j0 = g * _GRP
        for b in range(_GRP):
            pltpu.make_async_copy(
                out_hbm.at[pl.ds(0, _CHUNK)], bufs[b], gsems[b]).wait()
            pltpu.async_copy(
                bufs[b], out_hbm.at[pl.ds(base + (j0 + b) * _CHUNK, _CHUNK)],
                wsems[b])
        for b in range(_GRP):
            jj = j0 + b
            pltpu.make_async_copy(
                bufs[b], out_hbm.at[pl.ds(0, _CHUNK)], wsems[b]).wait()

            @pl.when(jj + _GRP < _NCH)
            def _():
                pltpu.async_copy(table_hbm.at[idx_v.at[jj + _GRP]],
                                 bufs[b], gsems[b])
        return carry

    lax.fori_loop(0, _NGRP, body, 0)


def _gather_rows(table, idx3):
    mesh = plsc.VectorSubcoreMesh(core_axis_name="c", subcore_axis_name="s")
    f = pl.kernel(
        _sc_gather_body,
        mesh=mesh,
        compiler_params=pltpu.CompilerParams(use_tc_tiling_on_sc=False),
        out_type=jax.ShapeDtypeStruct((_P, _C), jnp.float32),
        scratch_types=(
            [pltpu.VMEM((_NCH, _CHUNK), jnp.int32)]
            + [pltpu.VMEM((_CHUNK, _C), jnp.float32)] * _GRP
            + [pltpu.SemaphoreType.DMA] * (2 * _GRP)
        ),
    )
    return f(table, idx3)


def _tr_body(in_ref, out_ref):
    out_ref[...] = in_ref[...].T


def _transpose_rows(rows):
    qb = 768
    return pl.pallas_call(
        _tr_body,
        grid=(_P // qb,),
        in_specs=[pl.BlockSpec((qb, _C), lambda i: (i, 0))],
        out_specs=pl.BlockSpec((_C, qb), lambda i: (0, i)),
        out_shape=jax.ShapeDtypeStruct((_C, _P), jnp.float32),
    )(rows)


def kernel(features, lidar2ego, camera2ego, camera_intrinsics, img_aug_matrix,
           stride):
    pts = _points_zyx().reshape(3, _P)
    p = lidar2ego[:3, :3] @ pts + lidar2ego[:3, 3][:, None]
    p = jnp.broadcast_to(p[None], (_NCAM, 3, _P))
    p = p - camera2ego[:, :3, 3][:, :, None]
    p = jnp.linalg.inv(camera2ego[:, :3, :3]) @ p
    intr = camera_intrinsics.at[:, :2, :].set(
        camera_intrinsics[:, :2, :] / stride)
    p = intr @ p
    p = img_aug_matrix[:, :3, :3] @ p + img_aug_matrix[:, :3, 3][:, :, None]

    table = jnp.transpose(features, (0, 2, 3, 1)).reshape(_NROWS, _C)
    table = jnp.concatenate(
        [table, jnp.zeros((_TBL_ROWS - _NROWS, _C), jnp.float32)], axis=0)

    idx = _select_rows(p.reshape(_NCAM * 3, _P))
    idx3 = idx.reshape(_NW, _NCH, _CHUNK)
    rows = _gather_rows(table, idx3)
    out2 = _transpose_rows(rows)
    return out2.reshape(1, _C * _NZ, _NY, _NX)


# speedup vs baseline: 1.0020x; 1.0020x over previous
import jax
import jax.numpy as jnp
import numpy as np
from jax import lax
from jax.experimental import pallas as pl
from jax.experimental.pallas import tpu as pltpu
from jax.experimental.pallas import tpu_sc as plsc

_N_VOX = (200, 200, 12)
_VOXEL = np.array([0.5, 0.5, 8.0 / 12.0], dtype=np.float32)
_ORIGIN = np.array([0.0, 0.0, -1.0], dtype=np.float32)

_NCAM, _C, _H, _W = 6, 64, 64, 176
_NX, _NY, _NZ = _N_VOX
_P = _NX * _NY * _NZ
_NROWS = _NCAM * _H * _W
_ZROW = _NROWS
_TBL_ROWS = _NROWS + 8

_NC, _NS = 2, 16
_NW = _NC * _NS
_BPW = _P // _NW
_CHUNK = 120
_NCH = _BPW // _CHUNK
_GRP = 5
_NGRP = _NCH // _GRP

_QB = 19200
_NB = _P // _QB


def _points_zyx():
    zs = jnp.arange(_NZ, dtype=jnp.float32)
    ys = jnp.arange(_NY, dtype=jnp.float32)
    xs = jnp.arange(_NX, dtype=jnp.float32)
    zz, yy, xx = jnp.meshgrid(zs, ys, xs, indexing='ij')
    pts = jnp.stack([xx, yy, zz])
    vs = jnp.asarray(_VOXEL)
    new_origin = (jnp.asarray(_ORIGIN)
                  - jnp.asarray(_N_VOX, dtype=jnp.float32) / 2.0 * vs)
    return pts * vs.reshape(3, 1, 1, 1) + new_origin.reshape(3, 1, 1, 1)


def _sel_body(p_ref, idx_ref):
    s = jnp.full((1, _QB), _ZROW, dtype=jnp.int32)
    for i in range(_NCAM):
        px = p_ref[3 * i:3 * i + 1, :]
        py = p_ref[3 * i + 1:3 * i + 2, :]
        pz = p_ref[3 * i + 2:3 * i + 3, :]
        xi = jnp.round(px / pz).astype(jnp.int32)
        yi = jnp.round(py / pz).astype(jnp.int32)
        valid = ((xi >= 0) & (yi >= 0) & (xi < _W) & (yi < _H)
                 & (pz > 0.0))
        row = (yi * _W + xi) + i * (_H * _W)
        s = jnp.where(valid, row, s)
    idx_ref[...] = s[None]


def _select_rows(p2):
    return pl.pallas_call(
        _sel_body,
        grid=(_NB,),
        in_specs=[pl.BlockSpec((_NCAM * 3, _QB), lambda i: (0, i))],
        out_specs=pl.BlockSpec((1, 1, _QB), lambda i: (i, 0, 0)),
        out_shape=jax.ShapeDtypeStruct((_NB, 1, _QB), jnp.int32),
    )(p2)


def _sc_gather_body(table_hbm, idx_hbm, out_hbm, idx_v, *rest):
    bufs = rest[:_GRP]
    gsems = rest[_GRP:2 * _GRP]
    wsems = rest[2 * _GRP:]
    wid = lax.axis_index("s") * _NC + lax.axis_index("c")
    base = wid * _BPW

    pltpu.sync_copy(idx_hbm.at[wid], idx_v)

    for b in range(_GRP):
        pltpu.async_copy(table_hbm.at[idx_v.at[b]], bufs[b], gsems[b])

    def body(g, carry):
        j0 = g * _GRP
        for b in range(_GRP):
            pltpu.make_async_copy(
                out_hbm.at[pl.ds(0, _CHUNK)], bufs[b], gsems[b]).wait()
            pltpu.async_copy(
                bufs[b], out_hbm.at[pl.ds(base + (j0 + b) * _CHUNK, _CHUNK)],
                wsems[b])
        for b in range(_GRP):
            jj = j0 + b
            pltpu.make_async_copy(
                bufs[b], out_hbm.at[pl.ds(0, _CHUNK)], wsems[b]).wait()

            @pl.when(jj + _GRP < _NCH)
            def _():
                pltpu.async_copy(table_hbm.at[idx_v.at[jj + _GRP]],
                                 bufs[b], gsems[b])
        return carry

    lax.fori_loop(0, _NGRP, body, 0)


def _gather_rows(table, idx3):
    mesh = plsc.VectorSubcoreMesh(core_axis_name="c", subcore_axis_name="s")
    f = pl.kernel(
        _sc_gather_body,
        mesh=mesh,
        compiler_params=pltpu.CompilerParams(use_tc_tiling_on_sc=False),
        out_type=jax.ShapeDtypeStruct((_P, _C), jnp.float32),
        scratch_types=(
            [pltpu.VMEM((_NCH, _CHUNK), jnp.int32)]
            + [pltpu.VMEM((_CHUNK, _C), jnp.float32)] * _GRP
            + [pltpu.SemaphoreType.DMA] * (2 * _GRP)
        ),
    )
    return f(table, idx3)


def _tr_body(in_ref, out_ref):
    out_ref[...] = in_ref[...].T


def _transpose_rows(rows):
    qb = 768
    return pl.pallas_call(
        _tr_body,
        grid=(_P // qb,),
        in_specs=[pl.BlockSpec((qb, _C), lambda i: (i, 0))],
        out_specs=pl.BlockSpec((_C, qb), lambda i: (0, i)),
        out_shape=jax.ShapeDtypeStruct((_C, _P), jnp.float32),
    )(rows)


def kernel(features, lidar2ego, camera2ego, camera_intrinsics, img_aug_matrix,
           stride):
    pts = _points_zyx().reshape(3, _P)
    p = lidar2ego[:3, :3] @ pts + lidar2ego[:3, 3][:, None]
    p = jnp.broadcast_to(p[None], (_NCAM, 3, _P))
    p = p - camera2ego[:, :3, 3][:, :, None]
    p = jnp.linalg.inv(camera2ego[:, :3, :3]) @ p
    intr = camera_intrinsics.at[:, :2, :].set(
        camera_intrinsics[:, :2, :] / stride)
    p = intr @ p
    p = img_aug_matrix[:, :3, :3] @ p + img_aug_matrix[:, :3, 3][:, :, None]

    table = jnp.transpose(features, (0, 2, 3, 1)).reshape(_NROWS, _C)
    table = jnp.concatenate(
        [table, jnp.zeros((_TBL_ROWS - _NROWS, _C), jnp.float32)], axis=0)

    idx = _select_rows(p.reshape(_NCAM * 3, _P))
    idx3 = idx.reshape(_NW, _NCH, _CHUNK)
    rows = _gather_rows(table, idx3)
    out2 = _transpose_rows(rows)
    return out2.reshape(1, _C * _NZ, _NY, _NX)
